# baseline (speedup 1.0000x reference)
import functools

import jax
import jax.numpy as jnp
from jax import lax
from jax.experimental import pallas as pl
from jax.experimental.pallas import tpu as pltpu
from jax.experimental.pallas import tpu_sc as plsc

KK = 16
N = 1024
B = 4
NODES = B * N
GR = 2
GN = NODES // GR
CH_BLOCKS = [(3, 64), (64, 128), (128, 256)]


_NC, _NS, _L = 2, 16, 16
_NW = _NC * _NS


@functools.lru_cache(maxsize=None)
def _sc_gather_max(co, nodes):
    mesh = plsc.VectorSubcoreMesh(core_axis_name="c", subcore_axis_name="s")
    npw = nodes // _NW
    chunk = 8
    sub = 32
    nstream = (chunk * KK) // sub
    nchunk = npw // chunk

    @functools.partial(
        pl.kernel,
        out_type=jax.ShapeDtypeStruct((nodes, co), jnp.float32),
        mesh=mesh,
        scratch_types=[
            pltpu.VMEM((npw * KK,), jnp.int32),
            pltpu.VMEM((2, chunk * KK, co), jnp.float32),
            pltpu.VMEM((2, chunk, co), jnp.float32),
        ] + [pltpu.SemaphoreType.DMA] * 4,
    )
    def k(table_hbm, idx_hbm, out_hbm, idx_v, rows_v, out_v, *sems):
        wid = lax.axis_index("s") * _NC + lax.axis_index("c")
        base_node = wid * npw
        gsems = sems[:2]
        osems = sems[2:]

        pltpu.sync_copy(idx_hbm.at[pl.ds(base_node * KK, npw * KK)], idx_v)

        def fire(c, b):
            for u in range(nstream):
                pltpu.async_copy(
                    table_hbm.at[idx_v.at[pl.ds(c * chunk * KK + u * sub, sub)]],
                    rows_v.at[b, pl.ds(u * sub, sub)],
                    gsems[b])

        def wait_gather(b):
            for u in range(nstream):
                pltpu.make_async_copy(
                    table_hbm.at[idx_v.at[pl.ds(u * sub, sub)]],
                    rows_v.at[b, pl.ds(u * sub, sub)],
                    gsems[b]).wait()

        def compute(c, b):
            for i in range(chunk):
                for j in range(co // _L):
                    acc = rows_v[b, i * KK, pl.ds(j * _L, _L)]
                    for t in range(1, KK):
                        acc = jnp.maximum(acc, rows_v[b, i * KK + t, pl.ds(j * _L, _L)])
                    out_v[b, i, pl.ds(j * _L, _L)] = acc
            pltpu.async_copy(out_v.at[b],
                             out_hbm.at[pl.ds(base_node + c * chunk, chunk)],
                             osems[b])

        def wait_out(b):
            pltpu.make_async_copy(out_v.at[b],
                                  out_hbm.at[pl.ds(base_node, chunk)],
                                  osems[b]).wait()

        fire(0, 0)

        def pair_body(g, carry):
            c0 = 2 * g
            pl.when(c0 + 1 < nchunk)(lambda: fire(c0 + 1, 1))
            wait_gather(0)
            pl.when(g > 0)(lambda: wait_out(0))
            compute(c0, 0)
            pl.when(c0 + 2 < nchunk)(lambda: fire(c0 + 2, 0))
            pl.when(c0 + 1 < nchunk)(lambda: wait_gather(1))
            pl.when(g > 0)(lambda: wait_out(1))
            pl.when(c0 + 1 < nchunk)(lambda: compute(c0 + 1, 1))
            return carry

        lax.fori_loop(0, (nchunk + 1) // 2, pair_body, 0)
        wait_out(0)
        wait_out(1)

    return k


@functools.lru_cache(maxsize=None)
def _sc_gather_max_bf16(nodes):
    mesh = plsc.VectorSubcoreMesh(core_axis_name="c", subcore_axis_name="s")
    npw = nodes // _NW
    chunk = 8
    sub = 32
    nstream = (chunk * KK) // sub
    nchunk = npw // chunk

    @functools.partial(
        pl.kernel,
        out_type=jax.ShapeDtypeStruct((nodes, 128), jnp.int32),
        mesh=mesh,
        scratch_types=[
            pltpu.VMEM((npw * KK,), jnp.int32),
            pltpu.VMEM((2, chunk * KK, 128), jnp.int32),
            pltpu.VMEM((2, chunk, 128), jnp.int32),
            pltpu.SemaphoreType.DMA,
            pltpu.SemaphoreType.DMA,
            pltpu.SemaphoreType.DMA,
            pltpu.SemaphoreType.DMA,
        ],
    )
    def k(table_hbm, idx_hbm, out_hbm, idx_v, rows_v, out_v,
          gsem0, gsem1, osem0, osem1):
        wid = lax.axis_index("s") * _NC + lax.axis_index("c")
        base_node = wid * npw
        gsems = (gsem0, gsem1)
        osems = (osem0, osem1)

        pltpu.sync_copy(idx_hbm.at[pl.ds(base_node * KK, npw * KK)], idx_v)

        def fire(c, b):
            for u in range(nstream):
                pltpu.async_copy(
                    table_hbm.at[idx_v.at[pl.ds(c * chunk * KK + u * sub, sub)]],
                    rows_v.at[b, pl.ds(u * sub, sub)],
                    gsems[b])

        def wait_gather(b):
            for u in range(nstream):
                pltpu.make_async_copy(
                    table_hbm.at[idx_v.at[pl.ds(u * sub, sub)]],
                    rows_v.at[b, pl.ds(u * sub, sub)],
                    gsems[b]).wait()

        def compute(c, b):
            m16 = jnp.full((_L,), -65536, jnp.int32)
            s16 = jnp.full((_L,), 16, jnp.int32)

            def halves(w):
                lo = lax.bitcast_convert_type(jnp.left_shift(w, s16), jnp.float32)
                hi = lax.bitcast_convert_type(jnp.bitwise_and(w, m16), jnp.float32)
                return lo, hi

            for i in range(chunk):
                for h in range(8):
                    lo, hi = halves(rows_v[b, i * KK, pl.ds(h * _L, _L)])
                    for t in range(1, KK):
                        l2, h2 = halves(rows_v[b, i * KK + t, pl.ds(h * _L, _L)])
                        lo = jnp.maximum(lo, l2)
                        hi = jnp.maximum(hi, h2)
                    loi = lax.shift_right_logical(lax.bitcast_convert_type(lo, jnp.int32), s16)
                    hii = jnp.bitwise_and(lax.bitcast_convert_type(hi, jnp.int32), m16)
                    out_v[b, i, pl.ds(h * _L, _L)] = jnp.bitwise_or(loi, hii)
            pltpu.async_copy(out_v.at[b],
                             out_hbm.at[pl.ds(base_node + c * chunk, chunk)],
                             osems[b])

        def wait_out(b):
            pltpu.make_async_copy(out_v.at[b],
                                  out_hbm.at[pl.ds(base_node, chunk)],
                                  osems[b]).wait()

        fire(0, 0)

        def pair_body(g, carry):
            c0 = 2 * g
            pl.when(c0 + 1 < nchunk)(lambda: fire(c0 + 1, 1))
            wait_gather(0)
            pl.when(g > 0)(lambda: wait_out(0))
            compute(c0, 0)
            pl.when(c0 + 2 < nchunk)(lambda: fire(c0 + 2, 0))
            pl.when(c0 + 1 < nchunk)(lambda: wait_gather(1))
            pl.when(g > 0)(lambda: wait_out(1))
            pl.when(c0 + 1 < nchunk)(lambda: compute(c0 + 1, 1))
            return carry

        lax.fori_loop(0, (nchunk + 1) // 2, pair_body, 0)
        wait_out(0)
        wait_out(1)

    return k


def _mm(a, b):
    return jax.lax.dot_general(a, b, (((1,), (0,)), ((), ())),
                               preferred_element_type=jnp.float32)


def _sigmoid(z):
    return 1.0 / (1.0 + jnp.exp(-z))


def _knn_into(x_ref, idx_ref):
    nb = x_ref.shape[0] // N
    for b in range(nb):
        c = x_ref[pl.ds(b * N, N), :]
        sq = jnp.sum(c * c, axis=1)
        g = jax.lax.dot_general(c, c, (((1,), (1,)), ((), ())),
                                preferred_element_type=jnp.float32)
        negd = 2.0 * g - sq[:, None] - sq[None, :]
        colio = lax.broadcasted_iota(jnp.int32, (N, N), 1)
        lane16 = lax.broadcasted_iota(jnp.int32, (N, KK), 1)
        acc = jnp.zeros((N, KK), jnp.int32)
        for t in range(KK):
            am = jnp.argmax(negd, axis=1).astype(jnp.int32)
            acc = jnp.where(lane16 == t, (am + b * N)[:, None], acc)
            negd = jnp.where(colio == am[:, None], -jnp.inf, negd)
        idx_ref[pl.ds(b * N, N), :] = acc


_M16 = -65536


def _bf16_bits(x):
    b = lax.bitcast_convert_type(x, jnp.int32)
    r = b + (32767 + jnp.bitwise_and(lax.shift_right_logical(b, 16), 1))
    return jnp.bitwise_and(r, _M16)


def _pack_bf16(lo, hi):
    return jnp.bitwise_or(lax.shift_right_logical(_bf16_bits(lo), 16),
                          _bf16_bits(hi))


def _unpack_bf16(w):
    lo = lax.bitcast_convert_type(jnp.left_shift(w, 16), jnp.float32)
    hi = lax.bitcast_convert_type(jnp.bitwise_and(w, _M16), jnp.float32)
    return lo, hi


def _pre_into(feat, w_ref, b_ref, ci, pre1_ref, pre2_ref):
    w = w_ref[...]
    ci2, co = w.shape
    w1 = w[:ci, :]
    w2 = w[ci:, :]
    pre1_ref[...] = _mm(feat, w1 - w2) + b_ref[...]
    p2 = _mm(feat, w2)
    if pre2_ref.dtype == jnp.int32:
        pre2_ref[...] = _pack_bf16(p2[:, :128], p2[:, 128:])
    else:
        pad = pre2_ref.shape[1] - co
        if pad:
            p2 = jnp.concatenate(
                [p2, jnp.zeros((p2.shape[0], pad), jnp.float32)], axis=1)
        pre2_ref[...] = p2


def _se_scale(h, s1w_ref, s1b_ref, s2w_ref, s2b_ref):
    nb = h.shape[0] // N
    outs = []
    for b in range(nb):
        hb = h[b * N:(b + 1) * N, :]
        m = jnp.mean(hb, axis=0, keepdims=True)
        s1 = jnp.maximum(_mm(m, s1w_ref[...]) + s1b_ref[...], 0.0)
        s2 = _sigmoid(_mm(s1, s2w_ref[...]) + s2b_ref[...])
        outs.append(hb * s2)
    return jnp.concatenate(outs, axis=0)


def _head_body(x_ref, w_ref, b_ref, idx_ref, pre1_ref, pre2_ref):
    _knn_into(x_ref, idx_ref)
    _pre_into(x_ref[...], w_ref, b_ref, 3, pre1_ref, pre2_ref)


def _mid_body_s0(ci_next, pre1_ref, gm_ref, s1w, s1b, s2w, s2b,
                 nw_ref, nb_ref, feat_ref, pre1o_ref, pre2o_ref):
    co = pre1_ref.shape[1]
    h = jnp.maximum(pre1_ref[...] + gm_ref[...][:, :co], 0.0)
    feat = _se_scale(h, s1w, s1b, s2w, s2b)
    feat_ref[...] = feat
    _pre_into(feat, nw_ref, nb_ref, ci_next, pre1o_ref, pre2o_ref)


def _mid_body_sk(ci_next, pre1_ref, gm_ref, s1w, s1b, s2w, s2b,
                 pe_ref, ew, eb, pd_ref, e2w, e2b,
                 nw_ref, nb_ref, feat_ref, pre1o_ref, pre2o_ref):
    co = pre1_ref.shape[1]
    h = jnp.maximum(pre1_ref[...] + gm_ref[...][:, :co], 0.0)
    feat = _se_scale(h, s1w, s1b, s2w, s2b)
    feat = feat + jnp.maximum(_mm(pe_ref[...], ew[...]) + eb[...], 0.0)
    feat = feat + jnp.maximum(_mm(pd_ref[...], e2w[...]) + e2b[...], 0.0)
    feat_ref[...] = feat
    _pre_into(feat, nw_ref, nb_ref, ci_next, pre1o_ref, pre2o_ref)


def _gm_unpacked(gm_ref):
    lo, hi = _unpack_bf16(gm_ref[...])
    return jnp.concatenate([lo, hi], axis=1)


def _tail_a_s0(pre1_ref, gm_ref, s1w, s1b, s2w, s2b, f2_ref):
    h = jnp.maximum(pre1_ref[...] + _gm_unpacked(gm_ref), 0.0)
    f2_ref[...] = _se_scale(h, s1w, s1b, s2w, s2b)


def _tail_a_sk(pre1_ref, gm_ref, s1w, s1b, s2w, s2b,
               pe_ref, ew, eb, pd_ref, e2w, e2b, f2_ref):
    h = jnp.maximum(pre1_ref[...] + _gm_unpacked(gm_ref), 0.0)
    feat2 = _se_scale(h, s1w, s1b, s2w, s2b)
    feat2 = feat2 + jnp.maximum(_mm(pe_ref[...], ew[...]) + eb[...], 0.0)
    feat2 = feat2 + jnp.maximum(_mm(pd_ref[...], e2w[...]) + e2b[...], 0.0)
    f2_ref[...] = feat2


def _cat_fc(f0, f1, f2, catw, catb):
    w = catw[...]
    acc = _mm(f0, w[:64, :])
    acc = acc + _mm(f1, w[64:192, :])
    acc = acc + _mm(f2, w[192:, :])
    return jnp.maximum(acc + catb[...], 0.0)


def _gcat(a_ref, b_ref):
    return jnp.concatenate([a_ref[...], b_ref[...]], axis=0)


def _tail_b_s0(f0a, f0b, f1a, f1b, f2a, f2b, catw, catb, outf_ref):
    outf_ref[...] = _cat_fc(_gcat(f0a, f0b), _gcat(f1a, f1b), _gcat(f2a, f2b),
                            catw, catb)


def _tail_b_sk(f0a, f0b, f1a, f1b, f2a, f2b, catw, catb,
               poutf_ref, dw, db, fw, fb, outf_ref):
    catv = _cat_fc(_gcat(f0a, f0b), _gcat(f1a, f1b), _gcat(f2a, f2b),
                   catw, catb)
    dmix = jnp.maximum(_mm(poutf_ref[...], dw[...]) + db[...], 0.0)
    w = fw[...]
    fused = jnp.maximum(_mm(catv, w[:512, :]) + _mm(dmix, w[512:, :]) + fb[...], 0.0)
    outf_ref[...] = fused


def _tail_c(outf_ref, d0w, d0b, d1w, d1b, d2w, d2b, d3w, d3b, x_ref,
            xo_ref, h1_ref, h2_ref, h3_ref):
    h1 = jnp.maximum(_mm(outf_ref[...], d0w[...]) + d0b[...], 0.0)
    h2 = jnp.maximum(_mm(h1, d1w[...]) + d1b[...], 0.0)
    h3 = jnp.maximum(_mm(h2, d2w[...]) + d2b[...], 0.0)
    out = _mm(h3, d3w[...]) + d3b[...]
    h1_ref[...] = h1
    h2_ref[...] = h2
    h3_ref[...] = h3
    xo_ref[...] = x_ref[...] + out


def _f32(*shape):
    return jax.ShapeDtypeStruct(shape, jnp.float32)


def _tc_call(body, out_shape, name):
    return pl.pallas_call(body, out_shape=out_shape, name=name)


def kernel(x, params):
    p = params
    xf = x.reshape(NODES, 3)

    def b2(name):
        return p[name].reshape(1, -1)

    def grp(a, g):
        return a[g * GN:(g + 1) * GN]

    stage_x = xf
    prev_enc = prev_dec = prev_outf = None

    for s in range(4):
        co0 = CH_BLOCKS[0][1]
        idx = [None] * GR
        pre1 = [None] * GR
        pre2 = [None] * GR
        feats = [[None] * GR for _ in range(3)]
        for g in range(GR):
            idx[g], pre1[g], pre2[g] = _tc_call(
                _head_body,
                (jax.ShapeDtypeStruct((GN, KK), jnp.int32),
                 _f32(GN, co0), _f32(GN, max(co0, 128))),
                f"head{s}g{g}",
            )(grp(stage_x, g), p[f'g{s}0_W'], b2(f'g{s}0_b'))

        for g in range(GR):
            idx_flat = idx[g].reshape(-1)
            for blk in range(2):
                ci, co = CH_BLOCKS[blk]
                ci_next, co_next = CH_BLOCKS[blk + 1]
                gm = _sc_gather_max(pre2[g].shape[1], GN)(pre2[g], idx_flat)
                se = (p[f'g{s}{blk}_se1W'], b2(f'g{s}{blk}_se1b'),
                      p[f'g{s}{blk}_se2W'], b2(f'g{s}{blk}_se2b'))
                pre2_ty = (jax.ShapeDtypeStruct((GN, 128), jnp.int32)
                           if co_next == 256 else _f32(GN, co_next))
                outs = (_f32(GN, co), _f32(GN, co_next), pre2_ty)
                if s == 0:
                    feat, pre1[g], pre2[g] = _tc_call(
                        functools.partial(_mid_body_s0, ci_next),
                        outs, f"mid{s}_{blk}g{g}",
                    )(pre1[g], gm, *se, p[f'g{s}{blk+1}_W'], b2(f'g{s}{blk+1}_b'))
                else:
                    feat, pre1[g], pre2[g] = _tc_call(
                        functools.partial(_mid_body_sk, ci_next),
                        outs, f"mid{s}_{blk}g{g}",
                    )(pre1[g], gm, *se,
                      grp(prev_enc[blk], g),
                      p[f'E{s-1}_{blk}_W'], b2(f'E{s-1}_{blk}_b'),
                      grp(prev_dec[2 - blk], g),
                      p[f'E{s-1}_{5-blk}_W'], b2(f'E{s-1}_{5-blk}_b'),
                      p[f'g{s}{blk+1}_W'], b2(f'g{s}{blk+1}_b'))
                feats[blk][g] = feat

            gm = _sc_gather_max_bf16(GN)(pre2[g], idx_flat)
            se = (p[f'g{s}2_se1W'], b2(f'g{s}2_se1b'),
                  p[f'g{s}2_se2W'], b2(f'g{s}2_se2b'))
            if s == 0:
                feats[2][g] = _tc_call(_tail_a_s0, _f32(GN, 256), f"tailA{s}g{g}")(
                    pre1[g], gm, *se)
            else:
                feats[2][g] = _tc_call(_tail_a_sk, _f32(GN, 256), f"tailA{s}g{g}")(
                    pre1[g], gm, *se,
                    grp(prev_enc[2], g), p[f'E{s-1}_2_W'], b2(f'E{s-1}_2_b'),
                    grp(prev_dec[0], g), p[f'E{s-1}_3_W'], b2(f'E{s-1}_3_b'))

        flat_feats = [feats[blk][g] for blk in range(3) for g in range(GR)]
        if s == 0:
            outf = _tc_call(_tail_b_s0, _f32(NODES, 512), f"tailB{s}")(
                *flat_feats, p['cat0_W'], b2('cat0_b'))
        else:
            outf = _tc_call(_tail_b_sk, _f32(NODES, 512), f"tailB{s}")(
                *flat_feats, p[f'cat{s}_W'], b2(f'cat{s}_b'),
                prev_outf, p[f'D{s-1}_W'], b2(f'D{s-1}_b'),
                p[f'fuse{s}_W'], b2(f'fuse{s}_b'))
        dec_w = []
        for j in range(4):
            dec_w += [p[f'd{s}_{j}_W'], b2(f'd{s}_{j}_b')]
        stage_x, h1, h2, h3 = _tc_call(
            _tail_c,
            (_f32(NODES, 3), _f32(NODES, 256), _f32(NODES, 128), _f32(NODES, 64)),
            f"tailC{s}",
        )(outf, *dec_w, stage_x)
        enc = [jnp.concatenate([feats[blk][0], feats[blk][1]], axis=0)
               for blk in range(3)]
        prev_enc, prev_dec, prev_outf = enc, [h1, h2, h3], outf

    return stage_x.reshape(B, N, 3)


# speedup vs baseline: 1.0069x; 1.0069x over previous
import functools

import jax
import jax.numpy as jnp
from jax import lax
from jax.experimental import pallas as pl
from jax.experimental.pallas import tpu as pltpu
from jax.experimental.pallas import tpu_sc as plsc

KK = 16
N = 1024
B = 4
NODES = B * N
GR = 2
GN = NODES // GR
CH_BLOCKS = [(3, 64), (64, 128), (128, 256)]


_NC, _NS, _L = 2, 16, 16
_NW = _NC * _NS


@functools.lru_cache(maxsize=None)
def _sc_gather_max(co, nodes):
    mesh = plsc.VectorSubcoreMesh(core_axis_name="c", subcore_axis_name="s")
    npw = nodes // _NW
    chunk = 8
    sub = 32
    nstream = (chunk * KK) // sub
    nchunk = npw // chunk

    @functools.partial(
        pl.kernel,
        out_type=jax.ShapeDtypeStruct((nodes, co), jnp.float32),
        mesh=mesh,
        scratch_types=[
            pltpu.VMEM((npw * KK,), jnp.int32),
            pltpu.VMEM((2, chunk * KK, co), jnp.float32),
            pltpu.VMEM((2, chunk, co), jnp.float32),
        ] + [pltpu.SemaphoreType.DMA] * 4,
    )
    def k(table_hbm, idx_hbm, out_hbm, idx_v, rows_v, out_v, *sems):
        wid = lax.axis_index("s") * _NC + lax.axis_index("c")
        base_node = wid * npw
        gsems = sems[:2]
        osems = sems[2:]

        pltpu.sync_copy(idx_hbm.at[pl.ds(base_node * KK, npw * KK)], idx_v)

        def fire(c, b):
            for u in range(nstream):
                pltpu.async_copy(
                    table_hbm.at[idx_v.at[pl.ds(c * chunk * KK + u * sub, sub)]],
                    rows_v.at[b, pl.ds(u * sub, sub)],
                    gsems[b])

        def wait_gather(b):
            for u in range(nstream):
                pltpu.make_async_copy(
                    table_hbm.at[idx_v.at[pl.ds(u * sub, sub)]],
                    rows_v.at[b, pl.ds(u * sub, sub)],
                    gsems[b]).wait()

        def compute(c, b):
            for i in range(chunk):
                for j in range(co // _L):
                    acc = rows_v[b, i * KK, pl.ds(j * _L, _L)]
                    for t in range(1, KK):
                        acc = jnp.maximum(acc, rows_v[b, i * KK + t, pl.ds(j * _L, _L)])
                    out_v[b, i, pl.ds(j * _L, _L)] = acc
            pltpu.async_copy(out_v.at[b],
                             out_hbm.at[pl.ds(base_node + c * chunk, chunk)],
                             osems[b])

        def wait_out(b):
            pltpu.make_async_copy(out_v.at[b],
                                  out_hbm.at[pl.ds(base_node, chunk)],
                                  osems[b]).wait()

        fire(0, 0)

        def pair_body(g, carry):
            c0 = 2 * g
            pl.when(c0 + 1 < nchunk)(lambda: fire(c0 + 1, 1))
            wait_gather(0)
            pl.when(g > 0)(lambda: wait_out(0))
            compute(c0, 0)
            pl.when(c0 + 2 < nchunk)(lambda: fire(c0 + 2, 0))
            pl.when(c0 + 1 < nchunk)(lambda: wait_gather(1))
            pl.when(g > 0)(lambda: wait_out(1))
            pl.when(c0 + 1 < nchunk)(lambda: compute(c0 + 1, 1))
            return carry

        lax.fori_loop(0, (nchunk + 1) // 2, pair_body, 0)
        wait_out(0)
        wait_out(1)

    return k


@functools.lru_cache(maxsize=None)
def _sc_gather_max_bf16(nodes):
    mesh = plsc.VectorSubcoreMesh(core_axis_name="c", subcore_axis_name="s")
    npw = nodes // _NW
    chunk = 8
    sub = 32
    nstream = (chunk * KK) // sub
    nchunk = npw // chunk

    @functools.partial(
        pl.kernel,
        out_type=jax.ShapeDtypeStruct((nodes, 128), jnp.int32),
        mesh=mesh,
        scratch_types=[
            pltpu.VMEM((npw * KK,), jnp.int32),
            pltpu.VMEM((2, chunk * KK, 128), jnp.int32),
            pltpu.VMEM((2, chunk, 128), jnp.int32),
            pltpu.SemaphoreType.DMA,
            pltpu.SemaphoreType.DMA,
            pltpu.SemaphoreType.DMA,
            pltpu.SemaphoreType.DMA,
        ],
    )
    def k(table_hbm, idx_hbm, out_hbm, idx_v, rows_v, out_v,
          gsem0, gsem1, osem0, osem1):
        wid = lax.axis_index("s") * _NC + lax.axis_index("c")
        base_node = wid * npw
        gsems = (gsem0, gsem1)
        osems = (osem0, osem1)

        pltpu.sync_copy(idx_hbm.at[pl.ds(base_node * KK, npw * KK)], idx_v)

        def fire(c, b):
            for u in range(nstream):
                pltpu.async_copy(
                    table_hbm.at[idx_v.at[pl.ds(c * chunk * KK + u * sub, sub)]],
                    rows_v.at[b, pl.ds(u * sub, sub)],
                    gsems[b])

        def wait_gather(b):
            for u in range(nstream):
                pltpu.make_async_copy(
                    table_hbm.at[idx_v.at[pl.ds(u * sub, sub)]],
                    rows_v.at[b, pl.ds(u * sub, sub)],
                    gsems[b]).wait()

        def compute(c, b):
            m16 = jnp.full((_L,), -65536, jnp.int32)
            s16 = jnp.full((_L,), 16, jnp.int32)

            def halves(w):
                lo = lax.bitcast_convert_type(jnp.left_shift(w, s16), jnp.float32)
                hi = lax.bitcast_convert_type(jnp.bitwise_and(w, m16), jnp.float32)
                return lo, hi

            for i in range(chunk):
                for h in range(8):
                    lo, hi = halves(rows_v[b, i * KK, pl.ds(h * _L, _L)])
                    for t in range(1, KK):
                        l2, h2 = halves(rows_v[b, i * KK + t, pl.ds(h * _L, _L)])
                        lo = jnp.maximum(lo, l2)
                        hi = jnp.maximum(hi, h2)
                    loi = lax.shift_right_logical(lax.bitcast_convert_type(lo, jnp.int32), s16)
                    hii = jnp.bitwise_and(lax.bitcast_convert_type(hi, jnp.int32), m16)
                    out_v[b, i, pl.ds(h * _L, _L)] = jnp.bitwise_or(loi, hii)
            pltpu.async_copy(out_v.at[b],
                             out_hbm.at[pl.ds(base_node + c * chunk, chunk)],
                             osems[b])

        def wait_out(b):
            pltpu.make_async_copy(out_v.at[b],
                                  out_hbm.at[pl.ds(base_node, chunk)],
                                  osems[b]).wait()

        fire(0, 0)

        def pair_body(g, carry):
            c0 = 2 * g
            pl.when(c0 + 1 < nchunk)(lambda: fire(c0 + 1, 1))
            wait_gather(0)
            pl.when(g > 0)(lambda: wait_out(0))
            compute(c0, 0)
            pl.when(c0 + 2 < nchunk)(lambda: fire(c0 + 2, 0))
            pl.when(c0 + 1 < nchunk)(lambda: wait_gather(1))
            pl.when(g > 0)(lambda: wait_out(1))
            pl.when(c0 + 1 < nchunk)(lambda: compute(c0 + 1, 1))
            return carry

        lax.fori_loop(0, (nchunk + 1) // 2, pair_body, 0)
        wait_out(0)
        wait_out(1)

    return k


def _mm(a, b):
    return jax.lax.dot_general(a, b, (((1,), (0,)), ((), ())),
                               preferred_element_type=jnp.float32)


def _sigmoid(z):
    return 1.0 / (1.0 + jnp.exp(-z))


def _knn_into(x_ref, idx_ref):
    nb = x_ref.shape[0] // N
    for b in range(nb):
        c = x_ref[pl.ds(b * N, N), :]
        sq = jnp.sum(c * c, axis=1)
        g = jax.lax.dot_general(c, c, (((1,), (1,)), ((), ())),
                                preferred_element_type=jnp.float32)
        negd = 2.0 * g - sq[:, None] - sq[None, :]
        colio = lax.broadcasted_iota(jnp.int32, (N, N), 1)
        lane16 = lax.broadcasted_iota(jnp.int32, (N, KK), 1)
        acc = jnp.zeros((N, KK), jnp.int32)
        for t in range(KK):
            am = jnp.argmax(negd, axis=1).astype(jnp.int32)
            acc = jnp.where(lane16 == t, (am + b * N)[:, None], acc)
            negd = jnp.where(colio == am[:, None], -jnp.inf, negd)
        idx_ref[pl.ds(b * N, N), :] = acc


_M16 = -65536


def _bf16_bits(x):
    b = lax.bitcast_convert_type(x, jnp.int32)
    r = b + (32767 + jnp.bitwise_and(lax.shift_right_logical(b, 16), 1))
    return jnp.bitwise_and(r, _M16)


def _pack_bf16(lo, hi):
    return jnp.bitwise_or(lax.shift_right_logical(_bf16_bits(lo), 16),
                          _bf16_bits(hi))


def _unpack_bf16(w):
    lo = lax.bitcast_convert_type(jnp.left_shift(w, 16), jnp.float32)
    hi = lax.bitcast_convert_type(jnp.bitwise_and(w, _M16), jnp.float32)
    return lo, hi


def _pre_into(feat, w_ref, b_ref, ci, pre1_ref, pre2_ref):
    w = w_ref[...]
    ci2, co = w.shape
    w1 = w[:ci, :]
    w2 = w[ci:, :]
    pre1_ref[...] = _mm(feat, w1 - w2) + b_ref[...]
    p2 = _mm(feat, w2)
    if pre2_ref.dtype == jnp.int32:
        pre2_ref[...] = _pack_bf16(p2[:, :128], p2[:, 128:])
    else:
        pad = pre2_ref.shape[1] - co
        if pad:
            p2 = jnp.concatenate(
                [p2, jnp.zeros((p2.shape[0], pad), jnp.float32)], axis=1)
        pre2_ref[...] = p2


def _se_scale(h, s1w_ref, s1b_ref, s2w_ref, s2b_ref):
    nb = h.shape[0] // N
    outs = []
    for b in range(nb):
        hb = h[b * N:(b + 1) * N, :]
        m = jnp.mean(hb, axis=0, keepdims=True)
        s1 = jnp.maximum(_mm(m, s1w_ref[...]) + s1b_ref[...], 0.0)
        s2 = _sigmoid(_mm(s1, s2w_ref[...]) + s2b_ref[...])
        outs.append(hb * s2)
    return jnp.concatenate(outs, axis=0)


def _head_body(x_ref, w_ref, b_ref, idx_ref, pre1_ref, pre2_ref):
    _knn_into(x_ref, idx_ref)
    _pre_into(x_ref[...], w_ref, b_ref, 3, pre1_ref, pre2_ref)


def _mid_body_s0(ci_next, pre1_ref, gm_ref, s1w, s1b, s2w, s2b,
                 nw_ref, nb_ref, feat_ref, pre1o_ref, pre2o_ref):
    co = pre1_ref.shape[1]
    h = jnp.maximum(pre1_ref[...] + gm_ref[...][:, :co], 0.0)
    feat = _se_scale(h, s1w, s1b, s2w, s2b)
    feat_ref[...] = feat
    _pre_into(feat, nw_ref, nb_ref, ci_next, pre1o_ref, pre2o_ref)


def _mid_body_sk(ci_next, pre1_ref, gm_ref, s1w, s1b, s2w, s2b,
                 pe_ref, ew, eb, pd_ref, e2w, e2b,
                 nw_ref, nb_ref, feat_ref, pre1o_ref, pre2o_ref):
    co = pre1_ref.shape[1]
    h = jnp.maximum(pre1_ref[...] + gm_ref[...][:, :co], 0.0)
    feat = _se_scale(h, s1w, s1b, s2w, s2b)
    feat = feat + jnp.maximum(_mm(pe_ref[...], ew[...]) + eb[...], 0.0)
    feat = feat + jnp.maximum(_mm(pd_ref[...], e2w[...]) + e2b[...], 0.0)
    feat_ref[...] = feat
    _pre_into(feat, nw_ref, nb_ref, ci_next, pre1o_ref, pre2o_ref)


def _gm_unpacked(gm_ref):
    lo, hi = _unpack_bf16(gm_ref[...])
    return jnp.concatenate([lo, hi], axis=1)


def _tail_a_s0(pre1_ref, gm_ref, s1w, s1b, s2w, s2b, f2_ref):
    h = jnp.maximum(pre1_ref[...] + _gm_unpacked(gm_ref), 0.0)
    f2_ref[...] = _se_scale(h, s1w, s1b, s2w, s2b)


def _tail_a_sk(pre1_ref, gm_ref, s1w, s1b, s2w, s2b,
               pe_ref, ew, eb, pd_ref, e2w, e2b, f2_ref):
    h = jnp.maximum(pre1_ref[...] + _gm_unpacked(gm_ref), 0.0)
    feat2 = _se_scale(h, s1w, s1b, s2w, s2b)
    feat2 = feat2 + jnp.maximum(_mm(pe_ref[...], ew[...]) + eb[...], 0.0)
    feat2 = feat2 + jnp.maximum(_mm(pd_ref[...], e2w[...]) + e2b[...], 0.0)
    f2_ref[...] = feat2


def _cat_fc(f0, f1, f2, catw, catb):
    w = catw[...]
    acc = _mm(f0, w[:64, :])
    acc = acc + _mm(f1, w[64:192, :])
    acc = acc + _mm(f2, w[192:, :])
    return jnp.maximum(acc + catb[...], 0.0)


def _gcat(a_ref, b_ref):
    return jnp.concatenate([a_ref[...], b_ref[...]], axis=0)


def _tail_b_s0(f0a, f0b, f1a, f1b, f2a, f2b, catw, catb, outf_ref):
    outf_ref[...] = _cat_fc(_gcat(f0a, f0b), _gcat(f1a, f1b), _gcat(f2a, f2b),
                            catw, catb)


def _tail_b_sk(f0a, f0b, f1a, f1b, f2a, f2b, catw, catb,
               poutf_ref, dw, db, fw, fb, outf_ref):
    catv = _cat_fc(_gcat(f0a, f0b), _gcat(f1a, f1b), _gcat(f2a, f2b),
                   catw, catb)
    dmix = jnp.maximum(_mm(poutf_ref[...], dw[...]) + db[...], 0.0)
    w = fw[...]
    fused = jnp.maximum(_mm(catv, w[:512, :]) + _mm(dmix, w[512:, :]) + fb[...], 0.0)
    outf_ref[...] = fused


def _tail_c(outf_ref, d0w, d0b, d1w, d1b, d2w, d2b, d3w, d3b, x_ref,
            xo_ref, h1_ref, h2_ref, h3_ref):
    h1 = jnp.maximum(_mm(outf_ref[...], d0w[...]) + d0b[...], 0.0)
    h2 = jnp.maximum(_mm(h1, d1w[...]) + d1b[...], 0.0)
    h3 = jnp.maximum(_mm(h2, d2w[...]) + d2b[...], 0.0)
    out = _mm(h3, d3w[...]) + d3b[...]
    h1_ref[...] = h1
    h2_ref[...] = h2
    h3_ref[...] = h3
    xo_ref[...] = x_ref[...] + out


def _tail_c_last(outf_ref, d0w, d0b, d1w, d1b, d2w, d2b, d3w, d3b, x_ref,
                 xo_ref):
    h1 = jnp.maximum(_mm(outf_ref[...], d0w[...]) + d0b[...], 0.0)
    h2 = jnp.maximum(_mm(h1, d1w[...]) + d1b[...], 0.0)
    h3 = jnp.maximum(_mm(h2, d2w[...]) + d2b[...], 0.0)
    out = _mm(h3, d3w[...]) + d3b[...]
    xo_ref[...] = x_ref[...] + out


def _f32(*shape):
    return jax.ShapeDtypeStruct(shape, jnp.float32)


def _tc_call(body, out_shape, name):
    return pl.pallas_call(body, out_shape=out_shape, name=name)


def kernel(x, params):
    p = params
    xf = x.reshape(NODES, 3)

    def b2(name):
        return p[name].reshape(1, -1)

    def grp(a, g):
        return a[g * GN:(g + 1) * GN]

    stage_x = xf
    prev_enc = prev_dec = prev_outf = None

    for s in range(4):
        co0 = CH_BLOCKS[0][1]
        idx = [None] * GR
        pre1 = [None] * GR
        pre2 = [None] * GR
        feats = [[None] * GR for _ in range(3)]
        for g in range(GR):
            idx[g], pre1[g], pre2[g] = _tc_call(
                _head_body,
                (jax.ShapeDtypeStruct((GN, KK), jnp.int32),
                 _f32(GN, co0), _f32(GN, max(co0, 128))),
                f"head{s}g{g}",
            )(grp(stage_x, g), p[f'g{s}0_W'], b2(f'g{s}0_b'))

        for g in range(GR):
            idx_flat = idx[g].reshape(-1)
            for blk in range(2):
                ci, co = CH_BLOCKS[blk]
                ci_next, co_next = CH_BLOCKS[blk + 1]
                gm = _sc_gather_max(pre2[g].shape[1], GN)(pre2[g], idx_flat)
                se = (p[f'g{s}{blk}_se1W'], b2(f'g{s}{blk}_se1b'),
                      p[f'g{s}{blk}_se2W'], b2(f'g{s}{blk}_se2b'))
                pre2_ty = (jax.ShapeDtypeStruct((GN, 128), jnp.int32)
                           if co_next == 256 else _f32(GN, co_next))
                outs = (_f32(GN, co), _f32(GN, co_next), pre2_ty)
                if s == 0:
                    feat, pre1[g], pre2[g] = _tc_call(
                        functools.partial(_mid_body_s0, ci_next),
                        outs, f"mid{s}_{blk}g{g}",
                    )(pre1[g], gm, *se, p[f'g{s}{blk+1}_W'], b2(f'g{s}{blk+1}_b'))
                else:
                    feat, pre1[g], pre2[g] = _tc_call(
                        functools.partial(_mid_body_sk, ci_next),
                        outs, f"mid{s}_{blk}g{g}",
                    )(pre1[g], gm, *se,
                      grp(prev_enc[blk], g),
                      p[f'E{s-1}_{blk}_W'], b2(f'E{s-1}_{blk}_b'),
                      grp(prev_dec[2 - blk], g),
                      p[f'E{s-1}_{5-blk}_W'], b2(f'E{s-1}_{5-blk}_b'),
                      p[f'g{s}{blk+1}_W'], b2(f'g{s}{blk+1}_b'))
                feats[blk][g] = feat

            gm = _sc_gather_max_bf16(GN)(pre2[g], idx_flat)
            se = (p[f'g{s}2_se1W'], b2(f'g{s}2_se1b'),
                  p[f'g{s}2_se2W'], b2(f'g{s}2_se2b'))
            if s == 0:
                feats[2][g] = _tc_call(_tail_a_s0, _f32(GN, 256), f"tailA{s}g{g}")(
                    pre1[g], gm, *se)
            else:
                feats[2][g] = _tc_call(_tail_a_sk, _f32(GN, 256), f"tailA{s}g{g}")(
                    pre1[g], gm, *se,
                    grp(prev_enc[2], g), p[f'E{s-1}_2_W'], b2(f'E{s-1}_2_b'),
                    grp(prev_dec[0], g), p[f'E{s-1}_3_W'], b2(f'E{s-1}_3_b'))

        flat_feats = [feats[blk][g] for blk in range(3) for g in range(GR)]
        if s == 0:
            outf = _tc_call(_tail_b_s0, _f32(NODES, 512), f"tailB{s}")(
                *flat_feats, p['cat0_W'], b2('cat0_b'))
        else:
            outf = _tc_call(_tail_b_sk, _f32(NODES, 512), f"tailB{s}")(
                *flat_feats, p[f'cat{s}_W'], b2(f'cat{s}_b'),
                prev_outf, p[f'D{s-1}_W'], b2(f'D{s-1}_b'),
                p[f'fuse{s}_W'], b2(f'fuse{s}_b'))
        dec_w = []
        for j in range(4):
            dec_w += [p[f'd{s}_{j}_W'], b2(f'd{s}_{j}_b')]
        if s == 3:
            stage_x = _tc_call(_tail_c_last, _f32(NODES, 3), f"tailC{s}")(
                outf, *dec_w, stage_x)
        else:
            stage_x, h1, h2, h3 = _tc_call(
                _tail_c,
                (_f32(NODES, 3), _f32(NODES, 256), _f32(NODES, 128), _f32(NODES, 64)),
                f"tailC{s}",
            )(outf, *dec_w, stage_x)
            enc = [jnp.concatenate([feats[blk][0], feats[blk][1]], axis=0)
                   for blk in range(3)]
            prev_enc, prev_dec, prev_outf = enc, [h1, h2, h3], outf

    return stage_x.reshape(B, N, 3)
